# baseline (speedup 1.0000x reference)
import functools
import math

import jax
import jax.numpy as jnp
from jax import lax
from jax.experimental import pallas as pl
from jax.experimental.pallas import tpu as pltpu
from jax.experimental.pallas import tpu_sc as plsc

_D_MODEL = 100000
_D = 64
_SCALE = math.sqrt(_D_MODEL)
_NC, _NS = 2, 16
_NW = _NC * _NS
_NI, _NJ = 4096, 200
_B = _NI * _NJ
_IB = _NI // _NW
_B_PER_W = _IB * _NJ

_mesh = plsc.VectorSubcoreMesh(core_axis_name="c", subcore_axis_name="s")


@functools.partial(
    pl.kernel,
    out_type=jax.ShapeDtypeStruct((_NJ, 8, _NW, 8, 128), jnp.float32),
    mesh=_mesh,
    scratch_types=[
        pltpu.VMEM((_B_PER_W,), jnp.int32),
        pltpu.VMEM((_IB,), jnp.int32),
        pltpu.VMEM((_IB,), jnp.int32),
        pltpu.VMEM((_IB, _D), jnp.float32),
        pltpu.VMEM((_IB, _D), jnp.float32),
        pltpu.VMEM((8, 1, 8, 128), jnp.float32),
        pltpu.VMEM((8, 1, 8, 128), jnp.float32),
        pltpu.SemaphoreType.DMA,
        pltpu.SemaphoreType.DMA,
        pltpu.SemaphoreType.DMA,
        pltpu.SemaphoreType.DMA,
    ],
    compiler_params=pltpu.CompilerParams(
        use_tc_tiling_on_sc=False, needs_layout_passes=False),
)
def _embed_kernel(table_hbm, idx_hbm, out_hbm, idx_v, jidx0, jidx1,
                  rows0, rows1, slab0, slab1, gsem0, gsem1, osem0, osem1):
    wid = lax.axis_index("s") * _NC + lax.axis_index("c")
    base = wid * _B_PER_W
    jidx = (jidx0, jidx1)
    rows = (rows0, rows1)
    slab = (slab0, slab1)
    gsem = (gsem0, gsem1)
    osem = (osem0, osem1)

    pltpu.sync_copy(idx_hbm.at[pl.ds(base, _B_PER_W)], idx_v)

    iota = lax.iota(jnp.int32, 16)
    iota200 = iota * _NJ
    row_iota = [iota + (grp * 16) for grp in range(8)]

    def build_jidx(j, b):
        for grp in range(8):
            pos = iota200 + (grp * 16 * _NJ + j)
            vals = plsc.load_gather(idx_v, [pos])
            jidx[b][pl.ds(grp * 16, 16)] = vals

    def start_gather(b):
        pltpu.async_copy(table_hbm.at[jidx[b]], rows[b], gsem[b])

    def wait_gather(b):
        pltpu.make_async_copy(table_hbm.at[jidx[b]], rows[b], gsem[b]).wait()

    def transpose(b):
        def tk_body(tk, carry):
            for kk in range(8):
                col = jnp.full((16,), tk * 8 + kk, jnp.int32)
                for grp in range(8):
                    vals = plsc.load_gather(rows[b], [row_iota[grp], col])
                    slab[b][tk, 0, kk, pl.ds(grp * 16, 16)] = vals * _SCALE
            return carry
        lax.fori_loop(0, 8, tk_body, 0)

    def start_out(j, b):
        pltpu.async_copy(
            slab[b], out_hbm.at[j, :, pl.ds(wid, 1), :, :], osem[b])

    def wait_out(j, b):
        pltpu.make_async_copy(
            slab[b], out_hbm.at[j, :, pl.ds(wid, 1), :, :], osem[b]).wait()

    build_jidx(0, 0)
    start_gather(0)
    build_jidx(1, 1)
    start_gather(1)

    for j, b in ((0, 0), (1, 1)):
        wait_gather(b)
        transpose(b)
        build_jidx(j + 2, b)
        start_gather(b)
        start_out(j, b)

    def pair(k, carry):
        j = 2 * k
        for b in (0, 1):
            wait_gather(b)
            wait_out(j + b - 2, b)
            transpose(b)
            build_jidx(j + b + 2, b)
            start_gather(b)
            start_out(j + b, b)
        return carry

    lax.fori_loop(1, _NJ // 2 - 1, pair, 0)

    for j, b in ((_NJ - 2, 0), (_NJ - 1, 1)):
        wait_gather(b)
        wait_out(j - 2, b)
        transpose(b)
        start_out(j, b)
    wait_out(_NJ - 2, 0)
    wait_out(_NJ - 1, 1)


def kernel(x, lut):
    idx = x.reshape(-1).astype(jnp.int32)
    y = _embed_kernel(lut, idx)
    return y.transpose(2, 4, 0, 1, 3).reshape(_NI, _NJ, _D)


# speedup vs baseline: 2.3689x; 2.3689x over previous
import functools
import math

import jax
import jax.numpy as jnp
from jax import lax
from jax.experimental import pallas as pl
from jax.experimental.pallas import tpu as pltpu
from jax.experimental.pallas import tpu_sc as plsc

_D_MODEL = 100000
_D = 64
_SCALE = math.sqrt(_D_MODEL)
_NC, _NS = 2, 16
_NW = _NC * _NS
_NI, _NJ = 4096, 200
_B = _NI * _NJ
_IB = _NI // _NW
_B_PER_W = _IB * _NJ

_mesh = plsc.VectorSubcoreMesh(core_axis_name="c", subcore_axis_name="s")


@functools.partial(
    pl.kernel,
    out_type=jax.ShapeDtypeStruct((_NJ, 8, _NW, 1024), jnp.float32),
    mesh=_mesh,
    scratch_types=[
        pltpu.VMEM((_B_PER_W,), jnp.int32),
        pltpu.VMEM((_IB,), jnp.int32),
        pltpu.VMEM((_IB,), jnp.int32),
        pltpu.VMEM((_IB, _D), jnp.float32),
        pltpu.VMEM((_IB, _D), jnp.float32),
        pltpu.VMEM((8, 1, 1024), jnp.float32),
        pltpu.VMEM((8, 1, 1024), jnp.float32),
        pltpu.SemaphoreType.DMA,
        pltpu.SemaphoreType.DMA,
        pltpu.SemaphoreType.DMA,
        pltpu.SemaphoreType.DMA,
    ],
    compiler_params=pltpu.CompilerParams(
        use_tc_tiling_on_sc=False, needs_layout_passes=False),
)
def _embed_kernel(table_hbm, idx_hbm, out_hbm, idx_v, jidx0, jidx1,
                  rows0, rows1, slab0, slab1, gsem0, gsem1, osem0, osem1):
    wid = lax.axis_index("s") * _NC + lax.axis_index("c")
    base = wid * _B_PER_W
    jidx = (jidx0, jidx1)
    rows = (rows0, rows1)
    slab = (slab0, slab1)
    gsem = (gsem0, gsem1)
    osem = (osem0, osem1)

    pltpu.sync_copy(idx_hbm.at[pl.ds(base, _B_PER_W)], idx_v)

    iota = lax.iota(jnp.int32, 16)
    iota200 = iota * _NJ
    zerov = jnp.zeros((16,), jnp.int32)
    rot = [lax.rem(iota + r, 16) for r in range(16)]
    fp = [(r_v // 8) * 1024 + (r_v % 8) * 128 + iota for r_v in rot]

    def build_jidx(j, b):
        for grp in range(8):
            pos = iota200 + (grp * 16 * _NJ + j)
            vals = plsc.load_gather(idx_v, [pos])
            jidx[b][pl.ds(grp * 16, 16)] = vals

    def start_gather(b):
        pltpu.async_copy(table_hbm.at[jidx[b]], rows[b], gsem[b])

    def wait_gather(b):
        pltpu.make_async_copy(table_hbm.at[jidx[b]], rows[b], gsem[b]).wait()

    def transpose(b):
        def g_body(g, carry):
            ii0 = g * 16
            rowv = iota + ii0
            for q in range(4):
                k0 = q * 16
                cbase = k0 * 128 + ii0
                for r in range(16):
                    colv = rot[r] + k0
                    vals = plsc.load_gather(rows[b], [rowv, colv])
                    pos = fp[r] + cbase
                    plsc.store_scatter(
                        slab[b],
                        [lax.shift_right_logical(pos, 10), zerov,
                         lax.bitwise_and(pos, 1023)],
                        vals * _SCALE)
            return carry
        lax.fori_loop(0, 8, g_body, 0)

    def start_out(j, b):
        pltpu.async_copy(
            slab[b], out_hbm.at[j, :, pl.ds(wid, 1), :], osem[b])

    def wait_out(j, b):
        pltpu.make_async_copy(
            slab[b], out_hbm.at[j, :, pl.ds(wid, 1), :], osem[b]).wait()

    build_jidx(0, 0)
    start_gather(0)
    build_jidx(1, 1)
    start_gather(1)

    for j, b in ((0, 0), (1, 1)):
        wait_gather(b)
        transpose(b)
        build_jidx(j + 2, b)
        start_gather(b)
        start_out(j, b)

    def pair(k, carry):
        j = 2 * k
        for b in (0, 1):
            wait_gather(b)
            wait_out(j + b - 2, b)
            transpose(b)
            build_jidx(j + b + 2, b)
            start_gather(b)
            start_out(j + b, b)
        return carry

    lax.fori_loop(1, _NJ // 2 - 1, pair, 0)

    for j, b in ((_NJ - 2, 0), (_NJ - 1, 1)):
        wait_gather(b)
        wait_out(j - 2, b)
        transpose(b)
        start_out(j, b)
    wait_out(_NJ - 2, 0)
    wait_out(_NJ - 1, 1)


def kernel(x, lut):
    idx = x.reshape(-1).astype(jnp.int32)
    y = _embed_kernel(lut, idx)
    return (y.reshape(_NJ, 8, _NW, 8, 128)
             .transpose(2, 4, 0, 1, 3)
             .reshape(_NI, _NJ, _D))


# speedup vs baseline: 2.5708x; 1.0852x over previous
import functools
import math

import jax
import jax.numpy as jnp
from jax import lax
from jax.experimental import pallas as pl
from jax.experimental.pallas import tpu as pltpu
from jax.experimental.pallas import tpu_sc as plsc

_D_MODEL = 100000
_D = 64
_SCALE = math.sqrt(_D_MODEL)
_NC, _NS = 2, 16
_NW = _NC * _NS
_NI, _NJ = 4096, 200
_B = _NI * _NJ
_IB = _NI // _NW
_B_PER_W = _IB * _NJ

_mesh = plsc.VectorSubcoreMesh(core_axis_name="c", subcore_axis_name="s")


@functools.partial(
    pl.kernel,
    out_type=jax.ShapeDtypeStruct((_NJ, 8, _NW, 1024), jnp.float32),
    mesh=_mesh,
    scratch_types=[
        pltpu.VMEM((_B_PER_W,), jnp.int32),
        pltpu.VMEM((_IB,), jnp.int32),
        pltpu.VMEM((_IB,), jnp.int32),
        pltpu.VMEM((_IB, _D), jnp.float32),
        pltpu.VMEM((_IB, _D), jnp.float32),
        pltpu.VMEM((8 * 1024,), jnp.float32),
        pltpu.VMEM((8 * 1024,), jnp.float32),
        pltpu.SemaphoreType.DMA,
        pltpu.SemaphoreType.DMA,
        pltpu.SemaphoreType.DMA,
        pltpu.SemaphoreType.DMA,
    ],
    compiler_params=pltpu.CompilerParams(
        use_tc_tiling_on_sc=False, needs_layout_passes=False),
)
def _embed_kernel(table_hbm, idx_hbm, out_hbm, idx_v, jidx0, jidx1,
                  rows0, rows1, slab0, slab1, gsem0, gsem1, osem0, osem1):
    wid = lax.axis_index("s") * _NC + lax.axis_index("c")
    base = wid * _B_PER_W
    jidx = (jidx0, jidx1)
    rows = (rows0, rows1)
    slab = (slab0, slab1)
    gsem = (gsem0, gsem1)
    osem = (osem0, osem1)

    pltpu.sync_copy(idx_hbm.at[pl.ds(base, _B_PER_W)], idx_v)

    iota = lax.iota(jnp.int32, 16)
    iota200 = iota * _NJ
    rot = [lax.rem(iota + r, 16) for r in range(16)]
    ci = [iota + 16 * q for q in range(4)]
    fw = [(iota // 8) * 1024 + (iota % 8) * 128 + r_v for r_v in rot]

    def build_jidx(j, b):
        for grp in range(8):
            pos = iota200 + (grp * 16 * _NJ + j)
            vals = plsc.load_gather(idx_v, [pos])
            jidx[b][pl.ds(grp * 16, 16)] = vals

    def start_gather(b):
        pltpu.async_copy(table_hbm.at[jidx[b]], rows[b], gsem[b])

    def wait_gather(b):
        pltpu.make_async_copy(table_hbm.at[jidx[b]], rows[b], gsem[b]).wait()

    def transpose(b):
        def g_body(g, carry):
            ii0 = g * 16
            for q in range(4):
                wadd = q * 2048 + ii0
                for r in range(16):
                    vals = plsc.load_gather(rows[b], [rot[r] + ii0, ci[q]])
                    plsc.store_scatter(slab[b], [fw[r] + wadd], vals * _SCALE)
            return carry
        lax.fori_loop(0, 8, g_body, 0)

    def start_out(j, b):
        for tk in range(8):
            pltpu.async_copy(
                slab[b].at[pl.ds(tk * 1024, 1024)],
                out_hbm.at[j, tk, wid], osem[b])

    def wait_out(j, b):
        for tk in range(8):
            pltpu.make_async_copy(
                slab[b].at[pl.ds(tk * 1024, 1024)],
                out_hbm.at[j, tk, wid], osem[b]).wait()

    build_jidx(0, 0)
    start_gather(0)
    build_jidx(1, 1)
    start_gather(1)

    for j, b in ((0, 0), (1, 1)):
        wait_gather(b)
        transpose(b)
        build_jidx(j + 2, b)
        start_gather(b)
        start_out(j, b)

    def pair(k, carry):
        j = 2 * k
        for b in (0, 1):
            wait_gather(b)
            wait_out(j + b - 2, b)
            transpose(b)
            build_jidx(j + b + 2, b)
            start_gather(b)
            start_out(j + b, b)
        return carry

    lax.fori_loop(1, _NJ // 2 - 1, pair, 0)

    for j, b in ((_NJ - 2, 0), (_NJ - 1, 1)):
        wait_gather(b)
        wait_out(j - 2, b)
        transpose(b)
        start_out(j, b)
    wait_out(_NJ - 2, 0)
    wait_out(_NJ - 1, 1)


def kernel(x, lut):
    idx = x.reshape(-1).astype(jnp.int32)
    y = _embed_kernel(lut, idx)
    return (y.reshape(_NJ, 8, _NW, 8, 128)
             .transpose(2, 4, 0, 1, 3)
             .reshape(_NI, _NJ, _D))
